# baseline (speedup 1.0000x reference)
import functools

import jax
import jax.numpy as jnp
from jax import lax
from jax.experimental import pallas as pl
from jax.experimental.pallas import tpu as pltpu
from jax.experimental.pallas import tpu_sc as plsc

B, L, D = 8, 576, 64
K = 1024
N = B * L
DECAY = 0.99
TB = 512
NW = 32
TPW = N // NW
CNT_W = 16


def _assign_body(x_ref, cb_ref, idx_ref, sx2_ref):
    i = pl.program_id(0)
    x = x_ref[...]
    cb = cb_ref[...]
    xc = lax.dot_general(x, cb, (((1,), (1,)), ((), ())),
                         preferred_element_type=jnp.float32)
    x2 = jnp.sum(x * x, axis=1, keepdims=True)
    c2 = jnp.sum(cb * cb, axis=1)[None, :]
    d2 = x2 - 2.0 * xc + c2
    d = jnp.sqrt(jnp.maximum(d2, 0.0))
    dmin = jnp.min(d, axis=1, keepdims=True)
    kidx = lax.broadcasted_iota(jnp.int32, d.shape, 1)
    idx = jnp.min(jnp.where(d == dmin, kidx, K), axis=1)
    idx_ref[...] = idx.reshape(1, TB).astype(jnp.int32)

    @pl.when(i == 0)
    def _():
        sx2_ref[...] = jnp.zeros((1, 1), jnp.float32)
    sx2_ref[...] += jnp.sum(x * x).reshape(1, 1)


def _assign(x2d, codebook):
    return pl.pallas_call(
        _assign_body,
        grid=(N // TB,),
        in_specs=[
            pl.BlockSpec((TB, D), lambda i: (i, 0)),
            pl.BlockSpec((K, D), lambda i: (0, 0)),
        ],
        out_specs=[
            pl.BlockSpec((1, TB), lambda i: (0, i)),
            pl.BlockSpec((1, 1), lambda i: (0, 0)),
        ],
        out_shape=[
            jax.ShapeDtypeStruct((1, N), jnp.int32),
            jax.ShapeDtypeStruct((1, 1), jnp.float32),
        ],
    )(x2d, codebook)


@functools.cache
def _sc_kernels():
    mesh = plsc.VectorSubcoreMesh(core_axis_name="c", subcore_axis_name="s")
    cp = pltpu.CompilerParams(use_tc_tiling_on_sc=False)

    @functools.partial(
        pl.kernel,
        out_type=[
            jax.ShapeDtypeStruct((2, K, D), jnp.float32),
            jax.ShapeDtypeStruct((2, K, CNT_W), jnp.float32),
        ],
        mesh=mesh,
        scratch_types=[
            pltpu.VMEM((TPW,), jnp.int32),
            pltpu.VMEM((TPW, D), jnp.float32),
            pltpu.VMEM((TPW, CNT_W), jnp.float32),
            pltpu.VMEM_SHARED((K, D), jnp.float32),
            pltpu.VMEM_SHARED((K, CNT_W), jnp.float32),
        ],
        compiler_params=cp,
    )
    def _sc_scatter(x_hbm, idx_hbm, zdw_hbm, zcnt_hbm, ones_hbm,
                    pdw_hbm, pcnt_hbm,
                    idx_v, x_v, ones_v, acc_dw, acc_cnt):
        cid = lax.axis_index("c")
        sid = lax.axis_index("s")
        base = (sid * 2 + cid) * TPW

        @pl.when(sid == 0)
        def _():
            pltpu.sync_copy(zdw_hbm, acc_dw)
            pltpu.sync_copy(zcnt_hbm, acc_cnt)

        pltpu.sync_copy(idx_hbm.at[pl.ds(base, TPW)], idx_v)
        pltpu.sync_copy(x_hbm.at[pl.ds(base, TPW)], x_v)
        pltpu.sync_copy(ones_hbm, ones_v)
        plsc.subcore_barrier()

        pltpu.sync_copy(x_v, acc_dw.at[idx_v], add=True)
        pltpu.sync_copy(ones_v, acc_cnt.at[idx_v], add=True)
        plsc.subcore_barrier()

        @pl.when(sid == 0)
        def _():
            pltpu.sync_copy(acc_dw, pdw_hbm.at[cid])
            pltpu.sync_copy(acc_cnt, pcnt_hbm.at[cid])

    @functools.partial(
        pl.kernel,
        out_type=jax.ShapeDtypeStruct((N, D), jnp.float32),
        mesh=mesh,
        scratch_types=[
            pltpu.VMEM((TPW,), jnp.int32),
            pltpu.VMEM((TPW, D), jnp.float32),
            pltpu.SemaphoreType.DMA,
        ],
        compiler_params=cp,
    )
    def _sc_gather(cb_hbm, idx_hbm, out_hbm, idx_v, rows_v, sem):
        base = (lax.axis_index("s") * 2 + lax.axis_index("c")) * TPW
        pltpu.sync_copy(idx_hbm.at[pl.ds(base, TPW)], idx_v)
        pltpu.async_copy(cb_hbm.at[idx_v], rows_v, sem).wait()
        pltpu.sync_copy(rows_v, out_hbm.at[pl.ds(base, TPW)])

    return _sc_scatter, _sc_gather


def _update_body(counts_ref, ema_ref, pdw_ref, pcnt_ref, sx2_ref,
                 cnew_ref, enew_ref, cbnew_ref, loss_ref):
    hist = (pcnt_ref[0] + pcnt_ref[1])[:, 0:1]
    dw = pdw_ref[0] + pdw_ref[1]
    cnew = DECAY * counts_ref[...] + (1.0 - DECAY) * hist
    enew = DECAY * ema_ref[...] + (1.0 - DECAY) * dw
    cbnew = enew / cnew
    cnew_ref[...] = cnew
    enew_ref[...] = enew
    cbnew_ref[...] = cbnew
    cb2 = jnp.sum(cbnew * cbnew, axis=1, keepdims=True)
    loss = (0.5 * sx2_ref[...] - jnp.sum(cbnew * dw)
            + 0.5 * jnp.sum(hist * cb2)) / (N * D)
    loss_ref[...] = loss.reshape(1, 1)


def _update(counts2, ema_weight, pdw, pcnt, sx2):
    return pl.pallas_call(
        _update_body,
        out_shape=[
            jax.ShapeDtypeStruct((K, 1), jnp.float32),
            jax.ShapeDtypeStruct((K, D), jnp.float32),
            jax.ShapeDtypeStruct((K, D), jnp.float32),
            jax.ShapeDtypeStruct((1, 1), jnp.float32),
        ],
    )(counts2, ema_weight, pdw, pcnt, sx2)


def kernel(x, codebook, ema_weight, counts):
    sc_scatter, sc_gather = _sc_kernels()
    x2d = x.reshape(N, D)
    idx_row, sx2 = _assign(x2d, codebook)
    idx1d = idx_row.reshape(N)

    zdw = jnp.zeros((K, D), jnp.float32)
    zcnt = jnp.zeros((K, CNT_W), jnp.float32)
    ones = jnp.zeros((TPW, CNT_W), jnp.float32).at[:, 0].set(1.0)
    pdw, pcnt = sc_scatter(x2d, idx1d, zdw, zcnt, ones)

    cnew, enew, cbnew, loss = _update(counts.reshape(K, 1), ema_weight,
                                      pdw, pcnt, sx2)

    q2d = sc_gather(cbnew, idx1d)

    return (q2d.reshape(B, L, D), loss[0, 0], idx_row.reshape(B, L),
            cnew.reshape(K), enew, cbnew)


# speedup vs baseline: 1.2066x; 1.2066x over previous
import functools

import jax
import jax.numpy as jnp
from jax import lax
from jax.experimental import pallas as pl
from jax.experimental.pallas import tpu as pltpu
from jax.experimental.pallas import tpu_sc as plsc

B, L, D = 8, 576, 64
K = 1024
N = B * L
DECAY = 0.99
TB = 512
NB = N // TB
NW = 32
TPW = N // NW


def _fused_body(x_ref, cb_ref, counts_ref, ema_ref,
                idx_ref, cnew_ref, enew_ref, cbnew_ref, loss_ref,
                acc_dw, acc_hist, acc_sx2):
    i = pl.program_id(0)
    x = x_ref[...]
    cb = cb_ref[...]
    xc = lax.dot_general(x, cb, (((1,), (1,)), ((), ())),
                         preferred_element_type=jnp.float32)
    x2 = jnp.sum(x * x, axis=1, keepdims=True)
    c2 = jnp.sum(cb * cb, axis=1)[None, :]
    d = jnp.sqrt(jnp.maximum(x2 - 2.0 * xc + c2, 0.0))
    dmin = jnp.min(d, axis=1, keepdims=True)
    kidx = lax.broadcasted_iota(jnp.int32, d.shape, 1)
    idx = jnp.min(jnp.where(d == dmin, kidx, K), axis=1)
    idx_ref[...] = idx.reshape(1, TB).astype(jnp.int32)

    enc = (idx[:, None] == kidx).astype(jnp.float32)
    dw_blk = lax.dot_general(enc, x, (((0,), (0,)), ((), ())),
                             preferred_element_type=jnp.float32)
    hist_blk = lax.dot_general(enc, jnp.ones((TB, 1), jnp.float32),
                               (((0,), (0,)), ((), ())),
                               preferred_element_type=jnp.float32)

    @pl.when(i == 0)
    def _():
        acc_dw[...] = jnp.zeros((K, D), jnp.float32)
        acc_hist[...] = jnp.zeros((K, 1), jnp.float32)
        acc_sx2[...] = jnp.zeros((1, 1), jnp.float32)

    acc_dw[...] += dw_blk
    acc_hist[...] += hist_blk
    acc_sx2[...] += jnp.sum(x * x).reshape(1, 1)

    @pl.when(i == NB - 1)
    def _():
        hist = acc_hist[...]
        dw = acc_dw[...]
        cnew = DECAY * counts_ref[...] + (1.0 - DECAY) * hist
        enew = DECAY * ema_ref[...] + (1.0 - DECAY) * dw
        cbnew = enew / cnew
        cnew_ref[...] = cnew
        enew_ref[...] = enew
        cbnew_ref[...] = cbnew
        cb2 = jnp.sum(cbnew * cbnew, axis=1, keepdims=True)
        loss = (0.5 * acc_sx2[...] - jnp.sum(cbnew * dw)
                + 0.5 * jnp.sum(hist * cb2)) / (N * D)
        loss_ref[...] = loss.reshape(1, 1)


def _fused(x2d, codebook, counts2, ema_weight):
    return pl.pallas_call(
        _fused_body,
        grid=(NB,),
        in_specs=[
            pl.BlockSpec((TB, D), lambda i: (i, 0)),
            pl.BlockSpec((K, D), lambda i: (0, 0)),
            pl.BlockSpec((K, 1), lambda i: (0, 0)),
            pl.BlockSpec((K, D), lambda i: (0, 0)),
        ],
        out_specs=[
            pl.BlockSpec((1, TB), lambda i: (0, i)),
            pl.BlockSpec((K, 1), lambda i: (0, 0)),
            pl.BlockSpec((K, D), lambda i: (0, 0)),
            pl.BlockSpec((K, D), lambda i: (0, 0)),
            pl.BlockSpec((1, 1), lambda i: (0, 0)),
        ],
        out_shape=[
            jax.ShapeDtypeStruct((1, N), jnp.int32),
            jax.ShapeDtypeStruct((K, 1), jnp.float32),
            jax.ShapeDtypeStruct((K, D), jnp.float32),
            jax.ShapeDtypeStruct((K, D), jnp.float32),
            jax.ShapeDtypeStruct((1, 1), jnp.float32),
        ],
        scratch_shapes=[
            pltpu.VMEM((K, D), jnp.float32),
            pltpu.VMEM((K, 1), jnp.float32),
            pltpu.VMEM((1, 1), jnp.float32),
        ],
    )(x2d, codebook, counts2, ema_weight)


@functools.cache
def _sc_kernels():
    mesh = plsc.VectorSubcoreMesh(core_axis_name="c", subcore_axis_name="s")
    cp = pltpu.CompilerParams(use_tc_tiling_on_sc=False)

    @functools.partial(
        pl.kernel,
        out_type=jax.ShapeDtypeStruct((N, D), jnp.float32),
        mesh=mesh,
        scratch_types=[
            pltpu.VMEM((TPW,), jnp.int32),
            pltpu.VMEM((TPW, D), jnp.float32),
            pltpu.SemaphoreType.DMA,
        ],
        compiler_params=cp,
    )
    def _sc_gather(cb_hbm, idx_hbm, out_hbm, idx_v, rows_v, sem):
        base = (lax.axis_index("s") * 2 + lax.axis_index("c")) * TPW
        pltpu.sync_copy(idx_hbm.at[pl.ds(base, TPW)], idx_v)
        pltpu.async_copy(cb_hbm.at[idx_v], rows_v, sem).wait()
        pltpu.sync_copy(rows_v, out_hbm.at[pl.ds(base, TPW)])

    return _sc_gather


def kernel(x, codebook, ema_weight, counts):
    sc_gather = _sc_kernels()
    x2d = x.reshape(N, D)
    idx_row, cnew, enew, cbnew, loss = _fused(
        x2d, codebook, counts.reshape(K, 1), ema_weight)
    idx1d = idx_row.reshape(N)

    q2d = sc_gather(cbnew, idx1d)

    return (q2d.reshape(B, L, D), loss[0, 0], idx_row.reshape(B, L),
            cnew.reshape(K), enew, cbnew)


# speedup vs baseline: 1.8630x; 1.5440x over previous
import jax
import jax.numpy as jnp
from jax import lax
from jax.experimental import pallas as pl
from jax.experimental.pallas import tpu as pltpu

B, L, D = 8, 576, 64
K = 1024
N = B * L
DECAY = 0.99
TB = 1152
NBLK = N // TB


def _body(x_ref, cb_ref, counts_ref, ema_ref,
          q_ref, loss_ref, idx_ref, cnew_ref, enew_ref, cbnew_ref,
          enc_sc, acc_dw, acc_hist, acc_sx2, c2_sc):
    s = pl.program_id(0)

    @pl.when(s < NBLK)
    def _phase_a():
        x = x_ref[...]
        cb = cb_ref[...]

        @pl.when(s == 0)
        def _():
            c2_sc[...] = jnp.sum(cb * cb, axis=1, keepdims=True)
            acc_dw[...] = jnp.zeros((K, D), jnp.float32)
            acc_hist[...] = jnp.zeros((K, 1), jnp.float32)
            acc_sx2[...] = jnp.zeros((1, 1), jnp.float32)

        xc = lax.dot_general(cb, x, (((1,), (1,)), ((), ())),
                             preferred_element_type=jnp.float32)
        x2c = jnp.sum(x * x, axis=1, keepdims=True)
        x2r = jnp.transpose(x2c)
        d = jnp.sqrt(jnp.maximum(x2r - 2.0 * xc + c2_sc[...], 0.0))
        dmin = jnp.min(d, axis=0, keepdims=True)
        iota_sub = lax.broadcasted_iota(jnp.int32, (K, TB), 0)
        idx_ex = jnp.min(jnp.where(d == dmin, iota_sub, K), axis=0,
                         keepdims=True)
        enc_sc[...] = (iota_sub == idx_ex).astype(jnp.float32)
        idx_ref[:, pl.ds(s * TB, TB)] = idx_ex

        enc = enc_sc[...]
        acc_dw[...] += lax.dot_general(enc, x, (((1,), (0,)), ((), ())),
                                       preferred_element_type=jnp.float32)
        acc_hist[...] += lax.dot_general(
            enc, jnp.ones((TB, 1), jnp.float32), (((1,), (0,)), ((), ())),
            preferred_element_type=jnp.float32)
        acc_sx2[...] += jnp.sum(x2c).reshape(1, 1)

        @pl.when(s == NBLK - 1)
        def _update():
            hist = acc_hist[...]
            dw = acc_dw[...]
            cnt_col = jnp.transpose(counts_ref[...])
            cnew = DECAY * cnt_col + (1.0 - DECAY) * hist
            enew = DECAY * ema_ref[...] + (1.0 - DECAY) * dw
            cbnew = enew / cnew
            cnew_ref[...] = jnp.transpose(cnew)
            enew_ref[...] = enew
            cbnew_ref[...] = cbnew
            cb2 = jnp.sum(cbnew * cbnew, axis=1, keepdims=True)
            loss = (0.5 * acc_sx2[...] - jnp.sum(cbnew * dw)
                    + 0.5 * jnp.sum(hist * cb2)) / (N * D)
            loss_ref[...] = loss.reshape(1, 1)

    @pl.when(s >= NBLK)
    def _phase_b():
        t = s - NBLK
        iota_sub = lax.broadcasted_iota(jnp.int32, (K, TB), 0)
        idxr = idx_ref[:, pl.ds(t * TB, TB)]
        enc = (iota_sub == idxr).astype(jnp.float32)
        q = lax.dot_general(enc, cbnew_ref[...], (((0,), (0,)), ((), ())),
                            preferred_element_type=jnp.float32)
        q_ref[pl.ds(t * TB, TB), :] = q


def _fused(x2d, codebook, counts_row, ema_weight):
    return pl.pallas_call(
        _body,
        grid=(2 * NBLK,),
        in_specs=[
            pl.BlockSpec((TB, D), lambda s: (jnp.minimum(s, NBLK - 1), 0)),
            pl.BlockSpec((K, D), lambda s: (0, 0)),
            pl.BlockSpec((1, K), lambda s: (0, 0)),
            pl.BlockSpec((K, D), lambda s: (0, 0)),
        ],
        out_specs=[
            pl.BlockSpec((N, D), lambda s: (0, 0)),
            pl.BlockSpec((1, 1), lambda s: (0, 0)),
            pl.BlockSpec((1, N), lambda s: (0, 0)),
            pl.BlockSpec((1, K), lambda s: (0, 0)),
            pl.BlockSpec((K, D), lambda s: (0, 0)),
            pl.BlockSpec((K, D), lambda s: (0, 0)),
        ],
        out_shape=[
            jax.ShapeDtypeStruct((N, D), jnp.float32),
            jax.ShapeDtypeStruct((1, 1), jnp.float32),
            jax.ShapeDtypeStruct((1, N), jnp.int32),
            jax.ShapeDtypeStruct((1, K), jnp.float32),
            jax.ShapeDtypeStruct((K, D), jnp.float32),
            jax.ShapeDtypeStruct((K, D), jnp.float32),
        ],
        scratch_shapes=[
            pltpu.VMEM((K, TB), jnp.float32),
            pltpu.VMEM((K, D), jnp.float32),
            pltpu.VMEM((K, 1), jnp.float32),
            pltpu.VMEM((1, 1), jnp.float32),
            pltpu.VMEM((K, 1), jnp.float32),
        ],
    )(x2d, codebook, counts_row, ema_weight)


def kernel(x, codebook, ema_weight, counts):
    x2d = x.reshape(N, D)
    q2d, loss, idx, cnew, enew, cbnew = _fused(
        x2d, codebook, counts.reshape(1, K), ema_weight)
    return (q2d.reshape(B, L, D), loss[0, 0], idx.reshape(B, L),
            cnew.reshape(K), enew, cbnew)


# speedup vs baseline: 2.1581x; 1.1584x over previous
import jax
import jax.numpy as jnp
from jax import lax
from jax.experimental import pallas as pl
from jax.experimental.pallas import tpu as pltpu

B, L, D = 8, 576, 64
K = 1024
N = B * L
DECAY = 0.99
TB = 1152
NBLK = N // TB


def _body(x_ref, cb_ref, counts_ref, ema_ref,
          q_ref, loss_ref, idx_ref, cnew_ref, enew_ref, cbnew_ref,
          enc_sc, acc_dw, acc_hist, acc_sx2, c2_sc, cbm2_sc, rows_sc):
    s = pl.program_id(0)

    @pl.when(s < NBLK)
    def _phase_a():
        x = x_ref[...]

        @pl.when(s == 0)
        def _():
            cb = cb_ref[...]
            c2_sc[...] = jnp.sum(cb * cb, axis=1, keepdims=True)
            cbm2_sc[...] = -2.0 * cb
            acc_dw[...] = jnp.zeros((K, D), jnp.float32)
            acc_hist[...] = jnp.zeros((K, 1), jnp.float32)
            acc_sx2[...] = jnp.zeros((1, 1), jnp.float32)
            iota_k = lax.broadcasted_iota(jnp.int32, (8, K), 1)
            r = lax.broadcasted_iota(jnp.int32, (8, K), 0)
            rows = jnp.where(r == 0, 1.0,
                             jnp.where(r == 1, (iota_k // 32).astype(jnp.float32),
                                       jnp.where(r == 2,
                                                 (iota_k % 32).astype(jnp.float32),
                                                 0.0)))
            rows_sc[...] = rows.astype(jnp.float32)

        xcm2 = lax.dot_general(cbm2_sc[...], x, (((1,), (1,)), ((), ())),
                               preferred_element_type=jnp.float32)
        d = xcm2 + c2_sc[...]
        dmin = jnp.min(d, axis=0, keepdims=True)
        maskf = jnp.where(d == dmin, 1.0, 0.0).astype(jnp.float32)
        enc_sc[...] = maskf

        stat = lax.dot_general(rows_sc[...], maskf, (((1,), (0,)), ((), ())),
                               preferred_element_type=jnp.float32)
        idxf = 32.0 * stat[1:2, :] + stat[2:3, :]
        idx_ref[:, pl.ds(s * TB, TB)] = idxf.astype(jnp.int32)

        @pl.when(jnp.max(stat[0:1, :]) > 1.5)
        def _tie_fix():
            iota_sub = lax.broadcasted_iota(jnp.int32, (K, TB), 0)
            idx_ex = jnp.min(jnp.where(d == dmin, iota_sub, K), axis=0,
                             keepdims=True)
            enc_sc[...] = (iota_sub == idx_ex).astype(jnp.float32)
            idx_ref[:, pl.ds(s * TB, TB)] = idx_ex

        enc = enc_sc[...]
        acc_dw[...] += lax.dot_general(enc, x, (((1,), (0,)), ((), ())),
                                       preferred_element_type=jnp.float32)
        acc_hist[...] += lax.dot_general(
            enc, jnp.ones((TB, 1), jnp.float32), (((1,), (0,)), ((), ())),
            preferred_element_type=jnp.float32)
        acc_sx2[...] += jnp.sum(x * x).reshape(1, 1)

        @pl.when(s == NBLK - 1)
        def _update():
            hist = acc_hist[...]
            dw = acc_dw[...]
            cnt_col = jnp.transpose(counts_ref[...])
            cnew = DECAY * cnt_col + (1.0 - DECAY) * hist
            enew = DECAY * ema_ref[...] + (1.0 - DECAY) * dw
            cbnew = enew / cnew
            cnew_ref[...] = jnp.transpose(cnew)
            enew_ref[...] = enew
            cbnew_ref[...] = cbnew
            cb2 = jnp.sum(cbnew * cbnew, axis=1, keepdims=True)
            loss = (0.5 * acc_sx2[...] - jnp.sum(cbnew * dw)
                    + 0.5 * jnp.sum(hist * cb2)) / (N * D)
            loss_ref[...] = loss.reshape(1, 1)

    @pl.when(s >= NBLK)
    def _phase_b():
        t = s - NBLK
        iota_sub = lax.broadcasted_iota(jnp.int32, (K, TB), 0)
        idxr = idx_ref[:, pl.ds(t * TB, TB)]
        enc = (iota_sub == idxr).astype(jnp.float32)
        q = lax.dot_general(enc, cbnew_ref[...], (((0,), (0,)), ((), ())),
                            preferred_element_type=jnp.float32)
        q_ref[pl.ds(t * TB, TB), :] = q


def _fused(x2d, codebook, counts_row, ema_weight):
    return pl.pallas_call(
        _body,
        grid=(2 * NBLK,),
        in_specs=[
            pl.BlockSpec((TB, D), lambda s: (jnp.minimum(s, NBLK - 1), 0)),
            pl.BlockSpec((K, D), lambda s: (0, 0)),
            pl.BlockSpec((1, K), lambda s: (0, 0)),
            pl.BlockSpec((K, D), lambda s: (0, 0)),
        ],
        out_specs=[
            pl.BlockSpec((N, D), lambda s: (0, 0)),
            pl.BlockSpec((1, 1), lambda s: (0, 0)),
            pl.BlockSpec((1, N), lambda s: (0, 0)),
            pl.BlockSpec((1, K), lambda s: (0, 0)),
            pl.BlockSpec((K, D), lambda s: (0, 0)),
            pl.BlockSpec((K, D), lambda s: (0, 0)),
        ],
        out_shape=[
            jax.ShapeDtypeStruct((N, D), jnp.float32),
            jax.ShapeDtypeStruct((1, 1), jnp.float32),
            jax.ShapeDtypeStruct((1, N), jnp.int32),
            jax.ShapeDtypeStruct((1, K), jnp.float32),
            jax.ShapeDtypeStruct((K, D), jnp.float32),
            jax.ShapeDtypeStruct((K, D), jnp.float32),
        ],
        scratch_shapes=[
            pltpu.VMEM((K, TB), jnp.float32),
            pltpu.VMEM((K, D), jnp.float32),
            pltpu.VMEM((K, 1), jnp.float32),
            pltpu.VMEM((1, 1), jnp.float32),
            pltpu.VMEM((K, 1), jnp.float32),
            pltpu.VMEM((K, D), jnp.float32),
            pltpu.VMEM((8, K), jnp.float32),
        ],
    )(x2d, codebook, counts_row, ema_weight)


def kernel(x, codebook, ema_weight, counts):
    x2d = x.reshape(N, D)
    q2d, loss, idx, cnew, enew, cbnew = _fused(
        x2d, codebook, counts.reshape(1, K), ema_weight)
    return (q2d.reshape(B, L, D), loss[0, 0], idx.reshape(B, L),
            cnew.reshape(K), enew, cbnew)


# speedup vs baseline: 2.4115x; 1.1174x over previous
import jax
import jax.numpy as jnp
from jax import lax
from jax.experimental import pallas as pl
from jax.experimental.pallas import tpu as pltpu

B, L, D = 8, 576, 64
K = 1024
N = B * L
DECAY = 0.99
TB = 1152
NBLK = N // TB


def _body(x_ref, cb_ref, counts_ref, ema_ref,
          q_ref, loss_ref, idx_ref, cnew_ref, enew_ref, cbnew_ref,
          enc_sc, c2_sc, cbm2_sc, rows_sc, cbnb_sc, lacc_sc):
    s = pl.program_id(0)

    @pl.when(s < NBLK)
    def _phase_a():
        @pl.when(s == 0)
        def _init():
            cb = cb_ref[...]
            c2_sc[...] = jnp.sum(cb * cb, axis=1, keepdims=True)
            cbm2_sc[...] = -2.0 * cb
            iota_k = lax.broadcasted_iota(jnp.int32, (8, K), 1)
            r = lax.broadcasted_iota(jnp.int32, (8, K), 0)
            rows = jnp.where(
                r == 0, 1.0,
                jnp.where(r == 1, (iota_k // 32).astype(jnp.float32),
                          jnp.where(r == 2, (iota_k % 32).astype(jnp.float32),
                                    0.0)))
            rows_sc[...] = rows.astype(jnp.bfloat16)

        xs = x_ref[pl.ds(s * TB, TB), :]
        xcm2 = lax.dot_general(cbm2_sc[...], xs, (((1,), (1,)), ((), ())),
                               preferred_element_type=jnp.float32)
        d = xcm2 + c2_sc[...]
        dmin = jnp.min(d, axis=0, keepdims=True)
        maskb = jnp.where(d == dmin, 1.0, 0.0).astype(jnp.bfloat16)
        enc_sc[:, pl.ds(s * TB, TB)] = maskb

        stat = lax.dot_general(rows_sc[...], maskb, (((1,), (0,)), ((), ())),
                               preferred_element_type=jnp.float32)
        idxf = 32.0 * stat[1:2, :] + stat[2:3, :]
        idx_ref[:, pl.ds(s * TB, TB)] = idxf.astype(jnp.int32)

        @pl.when(jnp.max(stat[0:1, :]) > 1.5)
        def _tie_fix():
            iota_sub = lax.broadcasted_iota(jnp.int32, (K, TB), 0)
            idx_ex = jnp.min(jnp.where(d == dmin, iota_sub, K), axis=0,
                             keepdims=True)
            enc_sc[:, pl.ds(s * TB, TB)] = (iota_sub == idx_ex).astype(
                jnp.bfloat16)
            idx_ref[:, pl.ds(s * TB, TB)] = idx_ex

        @pl.when(s == NBLK - 1)
        def _update():
            xb = x_ref[...].astype(jnp.bfloat16)
            aug = jnp.concatenate(
                [xb, jnp.ones((N, 1), jnp.bfloat16),
                 jnp.zeros((N, 128 - D - 1), jnp.bfloat16)], axis=1)
            dw_aug = lax.dot_general(enc_sc[...], aug,
                                     (((1,), (0,)), ((), ())),
                                     preferred_element_type=jnp.float32)
            dw = dw_aug[:, :D]
            hist = dw_aug[:, D:D + 1]
            cnt_col = jnp.transpose(counts_ref[...])
            cnew = DECAY * cnt_col + (1.0 - DECAY) * hist
            enew = DECAY * ema_ref[...] + (1.0 - DECAY) * dw
            cbnew = enew / cnew
            cnew_ref[...] = jnp.transpose(cnew)
            enew_ref[...] = enew
            cbnew_ref[...] = cbnew
            cbnb_sc[...] = cbnew.astype(jnp.bfloat16)

    @pl.when(s >= NBLK)
    def _phase_b():
        t = s - NBLK

        @pl.when(s == NBLK)
        def _():
            lacc_sc[...] = jnp.zeros((1, D), jnp.float32)

        encb = enc_sc[:, pl.ds(t * TB, TB)]
        q = lax.dot_general(encb, cbnb_sc[...], (((0,), (0,)), ((), ())),
                            preferred_element_type=jnp.float32)
        q_ref[pl.ds(t * TB, TB), :] = q
        xs = x_ref[pl.ds(t * TB, TB), :]
        lacc_sc[...] += jnp.sum(jnp.square(xs - q), axis=0, keepdims=True)

        @pl.when(s == 2 * NBLK - 1)
        def _fin():
            loss_ref[...] = (0.5 * jnp.sum(lacc_sc[...])
                             / (N * D)).reshape(1, 1)


def _fused(x2d, codebook, counts_row, ema_weight):
    return pl.pallas_call(
        _body,
        grid=(2 * NBLK,),
        in_specs=[
            pl.BlockSpec((N, D), lambda s: (0, 0)),
            pl.BlockSpec((K, D), lambda s: (0, 0)),
            pl.BlockSpec((1, K), lambda s: (0, 0)),
            pl.BlockSpec((K, D), lambda s: (0, 0)),
        ],
        out_specs=[
            pl.BlockSpec((N, D), lambda s: (0, 0)),
            pl.BlockSpec((1, 1), lambda s: (0, 0)),
            pl.BlockSpec((1, N), lambda s: (0, 0)),
            pl.BlockSpec((1, K), lambda s: (0, 0)),
            pl.BlockSpec((K, D), lambda s: (0, 0)),
            pl.BlockSpec((K, D), lambda s: (0, 0)),
        ],
        out_shape=[
            jax.ShapeDtypeStruct((N, D), jnp.float32),
            jax.ShapeDtypeStruct((1, 1), jnp.float32),
            jax.ShapeDtypeStruct((1, N), jnp.int32),
            jax.ShapeDtypeStruct((1, K), jnp.float32),
            jax.ShapeDtypeStruct((K, D), jnp.float32),
            jax.ShapeDtypeStruct((K, D), jnp.float32),
        ],
        scratch_shapes=[
            pltpu.VMEM((K, N), jnp.bfloat16),
            pltpu.VMEM((K, 1), jnp.float32),
            pltpu.VMEM((K, D), jnp.float32),
            pltpu.VMEM((8, K), jnp.bfloat16),
            pltpu.VMEM((K, D), jnp.bfloat16),
            pltpu.VMEM((1, D), jnp.float32),
        ],
    )(x2d, codebook, counts_row, ema_weight)


def kernel(x, codebook, ema_weight, counts):
    x2d = x.reshape(N, D)
    q2d, loss, idx, cnew, enew, cbnew = _fused(
        x2d, codebook, counts.reshape(1, K), ema_weight)
    return (q2d.reshape(B, L, D), loss[0, 0], idx.reshape(B, L),
            cnew.reshape(K), enew, cbnew)


# speedup vs baseline: 2.4865x; 1.0311x over previous
import jax
import jax.numpy as jnp
from jax import lax
from jax.experimental import pallas as pl
from jax.experimental.pallas import tpu as pltpu

B, L, D = 8, 576, 64
K = 1024
N = B * L
DECAY = 0.99
TB = 1152
NBLK = N // TB


def _body(x_ref, cb_ref, counts_ref, ema_ref,
          q_ref, loss_ref, idx_ref, cnew_ref, enew_ref, cbnew_ref,
          enc_sc, c2_sc, cbm2_sc, rows_sc, cbnb_sc, lacc_sc):
    s = pl.program_id(0)

    @pl.when(s < NBLK)
    def _phase_a():
        @pl.when(s == 0)
        def _init():
            cb = cb_ref[...]
            c2_sc[...] = jnp.sum(cb * cb, axis=1, keepdims=True)
            cbm2_sc[...] = -2.0 * cb
            iota_k = lax.broadcasted_iota(jnp.int32, (8, K), 1)
            r = lax.broadcasted_iota(jnp.int32, (8, K), 0)
            rows = jnp.where(
                r == 0, 1.0,
                jnp.where(r == 1, (iota_k // 32).astype(jnp.float32),
                          jnp.where(r == 2, (iota_k % 32).astype(jnp.float32),
                                    0.0)))
            rows_sc[...] = rows.astype(jnp.bfloat16)

        xs = x_ref[pl.ds(s * TB, TB), :]
        xcm2 = lax.dot_general(cbm2_sc[...], xs, (((1,), (1,)), ((), ())),
                               preferred_element_type=jnp.float32)
        d = xcm2 + c2_sc[...]
        dmin = jnp.min(d, axis=0, keepdims=True)
        maskb = (d == dmin).astype(jnp.bfloat16)
        enc_sc[:, pl.ds(s * TB, TB)] = maskb

        stat = lax.dot_general(rows_sc[...], maskb, (((1,), (0,)), ((), ())),
                               preferred_element_type=jnp.float32)
        idxf = 32.0 * stat[1:2, :] + stat[2:3, :]
        idx_ref[:, pl.ds(s * TB, TB)] = idxf.astype(jnp.int32)

        @pl.when(jnp.max(stat[0:1, :]) > 1.5)
        def _tie_fix():
            iota_sub = lax.broadcasted_iota(jnp.int32, (K, TB), 0)
            idx_ex = jnp.min(jnp.where(d == dmin, iota_sub, K), axis=0,
                             keepdims=True)
            enc_sc[:, pl.ds(s * TB, TB)] = (iota_sub == idx_ex).astype(
                jnp.bfloat16)
            idx_ref[:, pl.ds(s * TB, TB)] = idx_ex

        @pl.when(s == NBLK - 1)
        def _update():
            xb = x_ref[...].astype(jnp.bfloat16)
            aug = jnp.concatenate(
                [xb, jnp.ones((N, 1), jnp.bfloat16),
                 jnp.zeros((N, 128 - D - 1), jnp.bfloat16)], axis=1)
            dw_aug = lax.dot_general(enc_sc[...], aug,
                                     (((1,), (0,)), ((), ())),
                                     preferred_element_type=jnp.float32)
            dw = dw_aug[:, :D]
            hist = dw_aug[:, D:D + 1]
            cnt_col = jnp.transpose(counts_ref[...])
            cnew = DECAY * cnt_col + (1.0 - DECAY) * hist
            enew = DECAY * ema_ref[...] + (1.0 - DECAY) * dw
            cbnew = enew / cnew
            cnew_ref[...] = jnp.transpose(cnew)
            enew_ref[...] = enew
            cbnew_ref[...] = cbnew
            cbnb_sc[...] = cbnew.astype(jnp.bfloat16)

    @pl.when(s == NBLK)
    def _phase_b():
        q = lax.dot_general(enc_sc[...], cbnb_sc[...],
                            (((0,), (0,)), ((), ())),
                            preferred_element_type=jnp.float32)
        q_ref[...] = q
        xs = x_ref[...]
        loss_ref[...] = (0.5 * jnp.sum(jnp.square(xs - q))
                         / (N * D)).reshape(1, 1)


def _fused(x2d, codebook, counts_row, ema_weight):
    return pl.pallas_call(
        _body,
        grid=(NBLK + 1,),
        in_specs=[
            pl.BlockSpec((N, D), lambda s: (0, 0)),
            pl.BlockSpec((K, D), lambda s: (0, 0)),
            pl.BlockSpec((1, K), lambda s: (0, 0)),
            pl.BlockSpec((K, D), lambda s: (0, 0)),
        ],
        out_specs=[
            pl.BlockSpec((N, D), lambda s: (0, 0)),
            pl.BlockSpec((1, 1), lambda s: (0, 0)),
            pl.BlockSpec((1, N), lambda s: (0, 0)),
            pl.BlockSpec((1, K), lambda s: (0, 0)),
            pl.BlockSpec((K, D), lambda s: (0, 0)),
            pl.BlockSpec((K, D), lambda s: (0, 0)),
        ],
        out_shape=[
            jax.ShapeDtypeStruct((N, D), jnp.float32),
            jax.ShapeDtypeStruct((1, 1), jnp.float32),
            jax.ShapeDtypeStruct((1, N), jnp.int32),
            jax.ShapeDtypeStruct((1, K), jnp.float32),
            jax.ShapeDtypeStruct((K, D), jnp.float32),
            jax.ShapeDtypeStruct((K, D), jnp.float32),
        ],
        scratch_shapes=[
            pltpu.VMEM((K, N), jnp.bfloat16),
            pltpu.VMEM((K, 1), jnp.float32),
            pltpu.VMEM((K, D), jnp.float32),
            pltpu.VMEM((8, K), jnp.bfloat16),
            pltpu.VMEM((K, D), jnp.bfloat16),
            pltpu.VMEM((1, D), jnp.float32),
        ],
    )(x2d, codebook, counts_row, ema_weight)


def kernel(x, codebook, ema_weight, counts):
    x2d = x.reshape(N, D)
    q2d, loss, idx, cnew, enew, cbnew = _fused(
        x2d, codebook, counts.reshape(1, K), ema_weight)
    return (q2d.reshape(B, L, D), loss[0, 0], idx.reshape(B, L),
            cnew.reshape(K), enew, cbnew)
